# baseline (speedup 1.0000x reference)
import jax
import jax.numpy as jnp
from jax.experimental import pallas as pl
from jax.experimental.pallas import tpu as pltpu

B = 16384
INPUT_DIM = 128
EMBED_DIM = 16
BLK = 8192
GRID = B // BLK
SUB = 512
NSUB = BLK // SUB


def _fused_kernel(enc_ref, floors_ref, emb_ref, w1_ref, w2_ref, b_ref, out_ref):
    p = jax.lax.dot_general(
        emb_ref[...], w2_ref[...],
        dimension_numbers=(((1,), (1,)), ((), ())),
        preferred_element_type=jnp.float32,
    )
    pdiff = p[1:2, :] - p[0:1, :]
    base = p[0:1, :] + b_ref[...]

    def body(s, carry):
        rows = pl.ds(s * SUB, SUB)
        dense = jax.lax.dot_general(
            enc_ref[rows, :], w1_ref[...],
            dimension_numbers=(((1,), (1,)), ((), ())),
            preferred_element_type=jnp.float32,
        )
        f = floors_ref[0, 0, rows].astype(jnp.float32)[:, None]
        out_ref[rows, :] = (dense + base) + f * pdiff
        return carry

    jax.lax.fori_loop(0, NSUB, body, 0)


def kernel(encodings, src_floors, emb_table, W, b):
    w1 = W[:, :INPUT_DIM]
    w2 = W[:, INPUT_DIM:]
    floors3 = src_floors.astype(jnp.int32).reshape(GRID, 1, BLK)
    b2 = b.reshape(1, INPUT_DIM)
    return pl.pallas_call(
        _fused_kernel,
        grid=(GRID,),
        in_specs=[
            pl.BlockSpec((BLK, INPUT_DIM), lambda i: (i, 0)),
            pl.BlockSpec((1, 1, BLK), lambda i: (i, 0, 0)),
            pl.BlockSpec((2, EMBED_DIM), lambda i: (0, 0)),
            pl.BlockSpec((INPUT_DIM, INPUT_DIM), lambda i: (0, 0)),
            pl.BlockSpec((INPUT_DIM, EMBED_DIM), lambda i: (0, 0)),
            pl.BlockSpec((1, INPUT_DIM), lambda i: (0, 0)),
        ],
        out_specs=pl.BlockSpec((BLK, INPUT_DIM), lambda i: (i, 0)),
        out_shape=jax.ShapeDtypeStruct((B, INPUT_DIM), jnp.float32),
        compiler_params=pltpu.CompilerParams(
            dimension_semantics=("arbitrary",),
        ),
    )(encodings, floors3, emb_table, w1, w2, b2)


# speedup vs baseline: 1.0105x; 1.0105x over previous
import jax
import jax.numpy as jnp
from jax.experimental import pallas as pl
from jax.experimental.pallas import tpu as pltpu

B = 16384
INPUT_DIM = 128
EMBED_DIM = 16
BLK = 8192
GRID = B // BLK
SUB = 512
NSUB = BLK // SUB


def _fused_kernel(enc_ref, floors_ref, emb_ref, w1_ref, w2_ref, b_ref, out_ref):
    p = jax.lax.dot_general(
        emb_ref[...], w2_ref[...],
        dimension_numbers=(((1,), (1,)), ((), ())),
        preferred_element_type=jnp.float32,
    )
    pdiff = p[1:2, :] - p[0:1, :]
    base = p[0:1, :] + b_ref[...]
    for s in range(NSUB):
        rows = pl.ds(s * SUB, SUB)
        dense = jax.lax.dot_general(
            enc_ref[rows, :], w1_ref[...],
            dimension_numbers=(((1,), (1,)), ((), ())),
            preferred_element_type=jnp.float32,
        )
        f = floors_ref[0, 0, rows].astype(jnp.float32)[:, None]
        out_ref[rows, :] = (dense + base) + f * pdiff


def kernel(encodings, src_floors, emb_table, W, b):
    w1 = W[:, :INPUT_DIM]
    w2 = W[:, INPUT_DIM:]
    floors3 = src_floors.astype(jnp.int32).reshape(GRID, 1, BLK)
    b2 = b.reshape(1, INPUT_DIM)
    return pl.pallas_call(
        _fused_kernel,
        grid=(GRID,),
        in_specs=[
            pl.BlockSpec((BLK, INPUT_DIM), lambda i: (i, 0)),
            pl.BlockSpec((1, 1, BLK), lambda i: (i, 0, 0)),
            pl.BlockSpec((2, EMBED_DIM), lambda i: (0, 0)),
            pl.BlockSpec((INPUT_DIM, INPUT_DIM), lambda i: (0, 0)),
            pl.BlockSpec((INPUT_DIM, EMBED_DIM), lambda i: (0, 0)),
            pl.BlockSpec((1, INPUT_DIM), lambda i: (0, 0)),
        ],
        out_specs=pl.BlockSpec((BLK, INPUT_DIM), lambda i: (i, 0)),
        out_shape=jax.ShapeDtypeStruct((B, INPUT_DIM), jnp.float32),
        input_output_aliases={0: 0},
        compiler_params=pltpu.CompilerParams(
            dimension_semantics=("arbitrary",),
        ),
    )(encodings, floors3, emb_table, w1, w2, b2)


# speedup vs baseline: 1.6880x; 1.6704x over previous
import jax
import jax.numpy as jnp
from jax.experimental import pallas as pl
from jax.experimental.pallas import tpu as pltpu

B = 16384
INPUT_DIM = 128
EMBED_DIM = 16
BLK = 8192
GRID = B // BLK
SUB = 512
NSUB = BLK // SUB


def _fused_kernel(enc_ref, floors_ref, emb_ref, w_ref, b_ref, out_ref):
    w2 = w_ref[:, INPUT_DIM:]
    p = jax.lax.dot_general(
        emb_ref[...], w2,
        dimension_numbers=(((1,), (1,)), ((), ())),
        preferred_element_type=jnp.float32,
    )
    pdiff = p[1:2, :] - p[0:1, :]
    base = p[0:1, :] + b_ref[...]
    w1 = w_ref[:, :INPUT_DIM]
    for s in range(NSUB):
        rows = pl.ds(s * SUB, SUB)
        dense = jax.lax.dot_general(
            enc_ref[rows, :], w1,
            dimension_numbers=(((1,), (1,)), ((), ())),
            preferred_element_type=jnp.float32,
        )
        f = floors_ref[0, 0, rows].astype(jnp.float32)[:, None]
        out_ref[rows, :] = (dense + base) + f * pdiff


def kernel(encodings, src_floors, emb_table, W, b):
    floors3 = src_floors.astype(jnp.int32).reshape(GRID, 1, BLK)
    b2 = b.reshape(1, INPUT_DIM)
    return pl.pallas_call(
        _fused_kernel,
        grid=(GRID,),
        in_specs=[
            pl.BlockSpec((BLK, INPUT_DIM), lambda i: (i, 0)),
            pl.BlockSpec((1, 1, BLK), lambda i: (i, 0, 0)),
            pl.BlockSpec((2, EMBED_DIM), lambda i: (0, 0)),
            pl.BlockSpec((INPUT_DIM, INPUT_DIM + EMBED_DIM), lambda i: (0, 0)),
            pl.BlockSpec((1, INPUT_DIM), lambda i: (0, 0)),
        ],
        out_specs=pl.BlockSpec((BLK, INPUT_DIM), lambda i: (i, 0)),
        out_shape=jax.ShapeDtypeStruct((B, INPUT_DIM), jnp.float32),
        compiler_params=pltpu.CompilerParams(
            dimension_semantics=("arbitrary",),
        ),
    )(encodings, floors3, emb_table, W, b2)
